# baseline (device time: 569896 ns/iter reference)
import jax
import jax.numpy as jnp
from jax import lax
from jax.experimental import pallas as pl
from jax.experimental.pallas import tpu as pltpu

M = 1024
K = 2048
V_HALF = 16384
CV = 1024
NC = V_HALF // CV
HM = M // 2
FWD_LAG = 2


def _fused(x, W):
    def body(
        x_ref,
        w_ref,
        e_ref,
        recv_buf,
        wv,
        lv,
        ov,
        rv,
        sv,
        sv_recv,
        w_sem,
        st_sem,
        ld_sem,
        s_send_sem,
        s_recv_sem,
        send_sems,
        recv_x_sems,
        fwd_sems,
        recv_y_sems,
    ):
        my_x = lax.axis_index("x")
        my_y = lax.axis_index("y")
        xnbr = (1 - my_x, my_y)
        ynbr = (my_x, 1 - my_y)
        my_base = my_x * V_HALF
        other_base = (1 - my_x) * V_HALF
        row_base = my_y * HM
        other_row = (1 - my_y) * HM

        barrier = pltpu.get_barrier_semaphore()
        for tgt in (xnbr, ynbr):
            pl.semaphore_signal(
                barrier, inc=1, device_id=tgt, device_id_type=pl.DeviceIdType.MESH
            )
        pl.semaphore_wait(barrier, 2)

        xv = x_ref[...]

        w_cp = [None] * NC
        w_cp[0] = pltpu.make_async_copy(
            w_ref.at[:, pl.ds(0, CV)], wv.at[0], w_sem.at[0]
        )
        w_cp[0].start()

        sends = [None] * NC
        fwds = [None] * NC
        stores = []
        s_acc = jnp.zeros((M, 1), jnp.float32)

        def emit_chunk(e, col_base, j_st):
            slot = j_st % 2
            if len(stores) >= 2:
                stores[-2].wait()
            ov[slot] = e
            st = pltpu.make_async_copy(
                ov.at[slot], e_ref.at[:, pl.ds(col_base, CV)], st_sem.at[slot]
            )
            st.start()
            stores.append(st)

        def wait_recv_x(j):
            pltpu.make_async_remote_copy(
                src_ref=lv.at[0, pl.ds(0, HM)],
                dst_ref=recv_buf.at[pl.ds(row_base, HM), pl.ds(j * CV, CV)],
                send_sem=send_sems.at[j],
                recv_sem=recv_x_sems.at[j],
                device_id=xnbr,
                device_id_type=pl.DeviceIdType.MESH,
            ).wait_recv()

        def start_fwd(j):
            fwds[j] = pltpu.make_async_remote_copy(
                src_ref=recv_buf.at[pl.ds(row_base, HM), pl.ds(j * CV, CV)],
                dst_ref=recv_buf.at[pl.ds(row_base, HM), pl.ds(j * CV, CV)],
                send_sem=fwd_sems.at[j],
                recv_sem=recv_y_sems.at[j],
                device_id=ynbr,
                device_id_type=pl.DeviceIdType.MESH,
            )
            fwds[j].start()

        for j in range(NC):
            slot = j % 2
            if j + 1 < NC:
                w_cp[j + 1] = pltpu.make_async_copy(
                    w_ref.at[:, pl.ds((j + 1) * CV, CV)],
                    wv.at[(j + 1) % 2],
                    w_sem.at[(j + 1) % 2],
                )
                w_cp[j + 1].start()
            w_cp[j].wait()
            if j >= 2:
                sends[j - 2].wait_send()
            l = jnp.dot(xv, wv[slot], preferred_element_type=jnp.float32)
            lv[slot] = l
            sends[j] = pltpu.make_async_remote_copy(
                src_ref=lv.at[slot, pl.ds(row_base, HM)],
                dst_ref=recv_buf.at[pl.ds(row_base, HM), pl.ds(j * CV, CV)],
                send_sem=send_sems.at[j],
                recv_sem=recv_x_sems.at[j],
                device_id=xnbr,
                device_id_type=pl.DeviceIdType.MESH,
            )
            sends[j].start()
            if j >= FWD_LAG:
                wait_recv_x(j - FWD_LAG)
                start_fwd(j - FWD_LAG)
            e = jnp.exp(l)
            s_acc = s_acc + jnp.sum(e, axis=1, keepdims=True)
            emit_chunk(e, my_base + j * CV, j)

        sv[...] = s_acc
        s_rdma = pltpu.make_async_remote_copy(
            src_ref=sv,
            dst_ref=sv_recv,
            send_sem=s_send_sem,
            recv_sem=s_recv_sem,
            device_id=xnbr,
            device_id_type=pl.DeviceIdType.MESH,
        )
        s_rdma.start()

        sends[NC - 2].wait_send()
        sends[NC - 1].wait_send()
        for j in range(NC - FWD_LAG, NC):
            wait_recv_x(j)
            start_fwd(j)

        s_rdma.wait()
        inv = 1.0 / (s_acc + sv_recv[...])

        for j in range(NC):
            pltpu.make_async_remote_copy(
                src_ref=lv.at[0, pl.ds(0, HM)],
                dst_ref=recv_buf.at[pl.ds(other_row, HM), pl.ds(j * CV, CV)],
                send_sem=send_sems.at[j],
                recv_sem=recv_y_sems.at[j],
                device_id=ynbr,
                device_id_type=pl.DeviceIdType.MESH,
            ).wait_recv()
            ld = pltpu.make_async_copy(
                recv_buf.at[:, pl.ds(j * CV, CV)], rv, ld_sem
            )
            ld.start()
            ldl = pltpu.make_async_copy(
                e_ref.at[:, pl.ds(my_base + j * CV, CV)],
                lv.at[j % 2],
                w_sem.at[j % 2],
            )
            ldl.start()
            ld.wait()
            e = jnp.exp(rv[...]) * inv
            emit_chunk(e, other_base + j * CV, NC + 2 * j)
            ldl.wait()
            n = lv[j % 2] * inv
            emit_chunk(n, my_base + j * CV, NC + 2 * j + 1)

        for j in range(NC):
            fwds[j].wait_send()
        stores[-2].wait()
        stores[-1].wait()

    out_shape = (
        jax.ShapeDtypeStruct((M, 2 * V_HALF), jnp.float32),
        jax.ShapeDtypeStruct((M, V_HALF), jnp.float32),
    )
    return pl.pallas_call(
        body,
        out_shape=out_shape,
        in_specs=[
            pl.BlockSpec(memory_space=pltpu.MemorySpace.VMEM),
            pl.BlockSpec(memory_space=pl.ANY),
        ],
        out_specs=(
            pl.BlockSpec(memory_space=pl.ANY),
            pl.BlockSpec(memory_space=pl.ANY),
        ),
        scratch_shapes=[
            pltpu.VMEM((2, K, CV), jnp.float32),
            pltpu.VMEM((2, M, CV), jnp.float32),
            pltpu.VMEM((2, M, CV), jnp.float32),
            pltpu.VMEM((M, CV), jnp.float32),
            pltpu.VMEM((M, 1), jnp.float32),
            pltpu.VMEM((M, 1), jnp.float32),
            pltpu.SemaphoreType.DMA((2,)),
            pltpu.SemaphoreType.DMA((2,)),
            pltpu.SemaphoreType.DMA,
            pltpu.SemaphoreType.DMA,
            pltpu.SemaphoreType.DMA,
            pltpu.SemaphoreType.DMA((NC,)),
            pltpu.SemaphoreType.DMA((NC,)),
            pltpu.SemaphoreType.DMA((NC,)),
            pltpu.SemaphoreType.DMA((NC,)),
        ],
        compiler_params=pltpu.CompilerParams(
            collective_id=0, vmem_limit_bytes=60 * 1024 * 1024
        ),
    )(x, W)


def kernel(x, W):
    out, _ = _fused(x, W)
    return out


# device time: 516721 ns/iter; 1.1029x vs baseline; 1.1029x over previous
import jax
import jax.numpy as jnp
from jax import lax
from jax.experimental import pallas as pl
from jax.experimental.pallas import tpu as pltpu

M = 1024
K = 2048
V_HALF = 16384
CV = 1024
NC = V_HALF // CV
HM = M // 2
FWD_LAG = 2
RLD_LAG = 4
RFIN_LAG = 5


def _fused(x, W):
    def body(
        x_ref,
        w_ref,
        e_ref,
        s_ref,
        recv_buf,
        wv,
        lv,
        ov,
        rv,
        w_sem,
        st_sem,
        ld_sem,
        send_sems,
        recv_x_sems,
        fwd_sems,
        recv_y_sems,
    ):
        my_x = lax.axis_index("x")
        my_y = lax.axis_index("y")
        xnbr = (1 - my_x, my_y)
        ynbr = (my_x, 1 - my_y)
        my_base = my_x * V_HALF
        other_base = (1 - my_x) * V_HALF
        row_base = my_y * HM
        other_row = (1 - my_y) * HM

        barrier = pltpu.get_barrier_semaphore()
        for tgt in (xnbr, ynbr):
            pl.semaphore_signal(
                barrier, inc=1, device_id=tgt, device_id_type=pl.DeviceIdType.MESH
            )
        pl.semaphore_wait(barrier, 2)

        xv = x_ref[...]

        w_cp = [None] * NC
        w_cp[0] = pltpu.make_async_copy(
            w_ref.at[:, pl.ds(0, CV)], wv.at[0], w_sem.at[0]
        )
        w_cp[0].start()

        sends = [None] * NC
        fwds = [None] * NC
        rlds = [None] * NC
        stores = []
        state = {"s": jnp.zeros((M, 1), jnp.float32)}

        def emit_chunk(e, col_base):
            slot = len(stores) % 2
            if len(stores) >= 2:
                stores[-2].wait()
            ov[slot] = e
            st = pltpu.make_async_copy(
                ov.at[slot], e_ref.at[:, pl.ds(col_base, CV)], st_sem.at[slot]
            )
            st.start()
            stores.append(st)

        def wait_recv_x(j):
            pltpu.make_async_remote_copy(
                src_ref=lv.at[0, pl.ds(0, HM)],
                dst_ref=recv_buf.at[pl.ds(row_base, HM), pl.ds(j * CV, CV)],
                send_sem=send_sems.at[j],
                recv_sem=recv_x_sems.at[j],
                device_id=xnbr,
                device_id_type=pl.DeviceIdType.MESH,
            ).wait_recv()

        def start_fwd(j):
            fwds[j] = pltpu.make_async_remote_copy(
                src_ref=recv_buf.at[pl.ds(row_base, HM), pl.ds(j * CV, CV)],
                dst_ref=recv_buf.at[pl.ds(row_base, HM), pl.ds(j * CV, CV)],
                send_sem=fwd_sems.at[j],
                recv_sem=recv_y_sems.at[j],
                device_id=ynbr,
                device_id_type=pl.DeviceIdType.MESH,
            )
            fwds[j].start()

        def start_remote_load(c):
            pltpu.make_async_remote_copy(
                src_ref=lv.at[0, pl.ds(0, HM)],
                dst_ref=recv_buf.at[pl.ds(other_row, HM), pl.ds(c * CV, CV)],
                send_sem=send_sems.at[c],
                recv_sem=recv_y_sems.at[c],
                device_id=ynbr,
                device_id_type=pl.DeviceIdType.MESH,
            ).wait_recv()
            rlds[c] = pltpu.make_async_copy(
                recv_buf.at[:, pl.ds(c * CV, CV)], rv.at[c % 2], ld_sem.at[c % 2]
            )
            rlds[c].start()

        def finish_remote(c):
            rlds[c].wait()
            e = jnp.exp(rv[c % 2])
            state["s"] = state["s"] + jnp.sum(e, axis=1, keepdims=True)
            emit_chunk(e, other_base + c * CV)

        for j in range(NC):
            slot = j % 2
            if j + 1 < NC:
                w_cp[j + 1] = pltpu.make_async_copy(
                    w_ref.at[:, pl.ds((j + 1) * CV, CV)],
                    wv.at[(j + 1) % 2],
                    w_sem.at[(j + 1) % 2],
                )
                w_cp[j + 1].start()
            w_cp[j].wait()
            if j >= 2:
                sends[j - 2].wait_send()
            l = jnp.dot(xv, wv[slot], preferred_element_type=jnp.float32)
            lv[slot] = l
            sends[j] = pltpu.make_async_remote_copy(
                src_ref=lv.at[slot, pl.ds(row_base, HM)],
                dst_ref=recv_buf.at[pl.ds(row_base, HM), pl.ds(j * CV, CV)],
                send_sem=send_sems.at[j],
                recv_sem=recv_x_sems.at[j],
                device_id=xnbr,
                device_id_type=pl.DeviceIdType.MESH,
            )
            sends[j].start()
            e = jnp.exp(lv[slot])
            state["s"] = state["s"] + jnp.sum(e, axis=1, keepdims=True)
            emit_chunk(e, my_base + j * CV)
            if j >= FWD_LAG:
                wait_recv_x(j - FWD_LAG)
                start_fwd(j - FWD_LAG)
            if j >= RLD_LAG:
                start_remote_load(j - RLD_LAG)
            if j >= RFIN_LAG:
                finish_remote(j - RFIN_LAG)

        sends[NC - 2].wait_send()
        sends[NC - 1].wait_send()
        for j in range(NC - FWD_LAG, NC):
            wait_recv_x(j)
            start_fwd(j)
        for c in range(NC - RLD_LAG, NC):
            start_remote_load(c)
            finish_remote(c - 1)
        finish_remote(NC - 1)
        for j in range(NC):
            fwds[j].wait_send()
        stores[-2].wait()
        stores[-1].wait()
        s_ref[...] = state["s"]

    out_shape = (
        jax.ShapeDtypeStruct((M, 2 * V_HALF), jnp.float32),
        jax.ShapeDtypeStruct((M, 1), jnp.float32),
        jax.ShapeDtypeStruct((M, V_HALF), jnp.float32),
    )
    return pl.pallas_call(
        body,
        out_shape=out_shape,
        in_specs=[
            pl.BlockSpec(memory_space=pltpu.MemorySpace.VMEM),
            pl.BlockSpec(memory_space=pl.ANY),
        ],
        out_specs=(
            pl.BlockSpec(memory_space=pl.ANY),
            pl.BlockSpec(memory_space=pltpu.MemorySpace.VMEM),
            pl.BlockSpec(memory_space=pl.ANY),
        ),
        scratch_shapes=[
            pltpu.VMEM((2, K, CV), jnp.float32),
            pltpu.VMEM((2, M, CV), jnp.float32),
            pltpu.VMEM((2, M, CV), jnp.float32),
            pltpu.VMEM((2, M, CV), jnp.float32),
            pltpu.SemaphoreType.DMA((2,)),
            pltpu.SemaphoreType.DMA((2,)),
            pltpu.SemaphoreType.DMA((2,)),
            pltpu.SemaphoreType.DMA((NC,)),
            pltpu.SemaphoreType.DMA((NC,)),
            pltpu.SemaphoreType.DMA((NC,)),
            pltpu.SemaphoreType.DMA((NC,)),
        ],
        compiler_params=pltpu.CompilerParams(
            collective_id=0, vmem_limit_bytes=63 * 1024 * 1024
        ),
    )(x, W)


def kernel(x, W):
    e, s, _ = _fused(x, W)
    return e / s
